# baseline (speedup 1.0000x reference)
import functools
import math

import jax
import jax.numpy as jnp
from jax import lax
from jax.experimental import pallas as pl
from jax.experimental.pallas import tpu as pltpu
from jax.experimental.pallas import tpu_sc as plsc

D_MODEL = 64
SCALE = math.sqrt(D_MODEL)
CHUNK = 256
NSTREAM = CHUNK // 128


def kernel(tokens, table):
    b, s = tokens.shape
    v, d = table.shape
    tok_phys = tokens.T.reshape(s, b // 128, 128).astype(jnp.int32)

    info = plsc.get_sparse_core_info()
    num_workers = info.num_cores * info.num_subcores
    nchunks = b // CHUNK

    mesh = plsc.VectorSubcoreMesh(core_axis_name="c", subcore_axis_name="s")

    @functools.partial(
        pl.kernel,
        mesh=mesh,
        out_type=jax.ShapeDtypeStruct((s, d, b), jnp.float32),
        scratch_types=[
            pltpu.VMEM((b // 128, 128), jnp.int32),
            pltpu.VMEM((2, CHUNK, d), jnp.float32),
            pltpu.VMEM((2, d, CHUNK), jnp.float32),
        ]
        + [pltpu.SemaphoreType.DMA] * 4,
        compiler_params=pltpu.CompilerParams(
            use_tc_tiling_on_sc=False, needs_layout_passes=False
        ),
    )
    def emb_kernel(tok_hbm, table_hbm, out_hbm, idx_sl, rows, out_t, *sems):
        gsems = sems[:2]
        osems = sems[2:]
        wid = lax.axis_index("s") * info.num_cores + lax.axis_index("c")
        nslices = (s - wid + num_workers - 1) // num_workers
        lane = jnp.arange(16, dtype=jnp.int32)

        def gather_start(j, p):
            for h in range(NSTREAM):
                pltpu.async_copy(
                    table_hbm.at[idx_sl.at[NSTREAM * j + h]],
                    rows.at[p, pl.ds(h * 128, 128)],
                    gsems[p],
                )

        def gather_wait(j, p):
            for h in range(NSTREAM):
                pltpu.make_async_copy(
                    table_hbm.at[idx_sl.at[NSTREAM * j + h]],
                    rows.at[p, pl.ds(h * 128, 128)],
                    gsems[p],
                ).wait()

        def write_start(sl, j, p):
            pltpu.async_copy(
                out_t.at[p], out_hbm.at[sl, :, pl.ds(j * CHUNK, CHUNK)], osems[p]
            )

        def write_wait(sl, j, p):
            pltpu.make_async_copy(
                out_t.at[p], out_hbm.at[sl, :, pl.ds(j * CHUNK, CHUNK)], osems[p]
            ).wait()

        def transpose_scale(p):
            def per_dim(c, carry):
                cvec = jnp.full((16,), c, dtype=jnp.int32)

                @plsc.parallel_loop(0, CHUNK // 16, unroll=4)
                def _(g):
                    ridx = g * 16 + lane
                    vals = plsc.load_gather(rows.at[p], [ridx, cvec])
                    out_t[p, c, pl.ds(g * 16, 16)] = vals * SCALE

                return carry

            lax.fori_loop(0, d, per_dim, 0)

        def slice_body(k, carry):
            sl = wid + k * num_workers
            pltpu.sync_copy(tok_hbm.at[sl], idx_sl)
            gather_start(0, 0)
            gather_start(1, 1)
            for j in range(nchunks):
                p = j % 2
                gather_wait(j, p)
                if j >= 2:
                    write_wait(sl, j - 2, p)
                transpose_scale(p)
                write_start(sl, j, p)
                if j + 2 < nchunks:
                    gather_start(j + 2, p)
            write_wait(sl, nchunks - 2, 0)
            write_wait(sl, nchunks - 1, 1)
            return carry

        lax.fori_loop(0, nslices, slice_body, 0)

    out_phys = emb_kernel(tok_phys, table)
    return jnp.transpose(out_phys, (2, 0, 1))


# speedup vs baseline: 1.8739x; 1.8739x over previous
import functools
import math

import jax
import jax.numpy as jnp
from jax import lax
from jax.experimental import pallas as pl
from jax.experimental.pallas import tpu as pltpu
from jax.experimental.pallas import tpu_sc as plsc

D_MODEL = 64
SCALE = math.sqrt(D_MODEL)
CHUNK = 256
NSTREAM = CHUNK // 128


def kernel(tokens, table):
    b, s = tokens.shape
    v, d = table.shape
    tok_phys = tokens.T.reshape(s, b // 128, 128).astype(jnp.int32)

    info = plsc.get_sparse_core_info()
    num_workers = info.num_cores * info.num_subcores
    nchunks = b // CHUNK

    mesh = plsc.VectorSubcoreMesh(core_axis_name="c", subcore_axis_name="s")

    @functools.partial(
        pl.kernel,
        mesh=mesh,
        out_type=jax.ShapeDtypeStruct((s, d, b), jnp.float32),
        scratch_types=[
            pltpu.VMEM((b // 128, 128), jnp.int32),
            pltpu.VMEM((2, CHUNK, d), jnp.float32),
            pltpu.VMEM((2, d, CHUNK), jnp.float32),
        ]
        + [pltpu.SemaphoreType.DMA] * 4,
        compiler_params=pltpu.CompilerParams(
            use_tc_tiling_on_sc=False, needs_layout_passes=False
        ),
    )
    def emb_kernel(tok_hbm, table_hbm, out_hbm, idx_sl, rows, out_t, *sems):
        gsems = sems[:2]
        osems = sems[2:]
        wid = lax.axis_index("s") * info.num_cores + lax.axis_index("c")
        nslices = (s - wid + num_workers - 1) // num_workers
        lane = jnp.arange(16, dtype=jnp.int32)

        def gather_start(j, p):
            for h in range(NSTREAM):
                pltpu.async_copy(
                    table_hbm.at[idx_sl.at[NSTREAM * j + h]],
                    rows.at[p, pl.ds(h * 128, 128)],
                    gsems[p],
                )

        def gather_wait(j, p):
            for h in range(NSTREAM):
                pltpu.make_async_copy(
                    table_hbm.at[idx_sl.at[NSTREAM * j + h]],
                    rows.at[p, pl.ds(h * 128, 128)],
                    gsems[p],
                ).wait()

        def write_start(sl, j, p):
            pltpu.async_copy(
                out_t.at[p], out_hbm.at[sl, :, pl.ds(j * CHUNK, CHUNK)], osems[p]
            )

        def write_wait(sl, j, p):
            pltpu.make_async_copy(
                out_t.at[p], out_hbm.at[sl, :, pl.ds(j * CHUNK, CHUNK)], osems[p]
            ).wait()

        def transpose_scale(p):
            for cb in range(d // 16):
                cbase = cb * 16

                @plsc.parallel_loop(0, (CHUNK // 16) * 16, unroll=4)
                def _(t):
                    rb = t >> 4
                    dd = t & 15
                    ridx = rb * 16 + lane
                    crot = cbase + ((lane + dd) & 15)
                    vals = plsc.load_gather(rows.at[p], [ridx, crot])
                    plsc.store_scatter(out_t.at[p], [crot, ridx], vals * SCALE)

        def slice_body(k, carry):
            sl = wid + k * num_workers
            pltpu.sync_copy(tok_hbm.at[sl], idx_sl)
            gather_start(0, 0)
            gather_start(1, 1)
            for j in range(nchunks):
                p = j % 2
                gather_wait(j, p)
                if j >= 2:
                    write_wait(sl, j - 2, p)
                transpose_scale(p)
                write_start(sl, j, p)
                if j + 2 < nchunks:
                    gather_start(j + 2, p)
            write_wait(sl, nchunks - 2, 0)
            write_wait(sl, nchunks - 1, 1)
            return carry

        lax.fori_loop(0, nslices, slice_body, 0)

    out_phys = emb_kernel(tok_phys, table)
    return jnp.transpose(out_phys, (2, 0, 1))
